# baseline (speedup 1.0000x reference)
import functools

import jax
import jax.numpy as jnp
from jax import lax
from jax.experimental import pallas as pl
from jax.experimental.pallas import tpu as pltpu
from jax.experimental.pallas import tpu_sc as plsc

B = 4
S = 8192
D = 1024

_INFO = plsc.get_sparse_core_info()
NC = _INFO.num_cores
NS = _INFO.num_subcores
NW = NC * NS
LANES = 16

ROWS_PER_W = S // NW
C = 32
CL = C * D
N_CHUNKS = ROWS_PER_W // C
BATCH_STRIDE = S * D

B_SC = 2
B_TC = B - B_SC
TC_CS = 512


def _add_chunk(iobuf, tbuf):
    @plsc.parallel_loop(0, C, step=1)
    def _add_row(r):
        @plsc.parallel_loop(0, D, step=LANES, unroll=8)
        def _add(j):
            plsc.addupdate(iobuf.at[r, pl.ds(j, LANES)], tbuf[r, pl.ds(j, LANES)])


def _body(in_hbm, tab_hbm, out_hbm, tbuf, io0, io1, sem_in, sem_out, sem_tab):
    cid = lax.axis_index("c")
    sid = lax.axis_index("s")
    wid = sid * NC + cid
    w_row0 = wid * ROWS_PER_W

    ios = (io0, io1)
    steps = [(c, b) for c in range(N_CHUNKS) for b in range(B_SC)]
    n = len(steps)

    def in_copy(s, buf):
        c, b = steps[s]
        r0 = w_row0 + c * C
        return pltpu.async_copy(in_hbm.at[B_TC + b, pl.ds(r0, C)], buf, sem_in)

    tab_dma = pltpu.async_copy(tab_hbm.at[pl.ds(w_row0, C)], tbuf, sem_tab)
    in_dma = in_copy(0, ios[0])
    out_pending = [None, None]

    for s, (c, b) in enumerate(steps):
        p = s % 2
        if b == 0:
            tab_dma.wait()
        if s + 1 < n:
            if out_pending[1 - p] is not None:
                out_pending[1 - p].wait()
            nxt = in_copy(s + 1, ios[1 - p])
        in_dma.wait()
        in_dma = nxt if s + 1 < n else None
        _add_chunk(ios[p], tbuf)
        if b == B_SC - 1 and c + 1 < N_CHUNKS:
            tab_dma = pltpu.async_copy(
                tab_hbm.at[pl.ds(w_row0 + (c + 1) * C, C)], tbuf, sem_tab
            )
        r0 = w_row0 + c * C
        out_pending[p] = pltpu.async_copy(ios[p], out_hbm.at[b, pl.ds(r0, C)], sem_out)

    for d in out_pending:
        if d is not None:
            d.wait()


def _tc_body(tab_ref, in_ref, out_ref):
    out_ref[...] = in_ref[...] + tab_ref[...][None]


@jax.jit
def kernel(input_embeddings, emb_table):
    kfn = pl.kernel(
        _body,
        out_type=jax.ShapeDtypeStruct((B_SC, S, D), jnp.float32),
        mesh=plsc.VectorSubcoreMesh(core_axis_name="c", subcore_axis_name="s"),
        scratch_types=[
            pltpu.VMEM((C, D), jnp.float32),
            pltpu.VMEM((C, D), jnp.float32),
            pltpu.VMEM((C, D), jnp.float32),
            pltpu.SemaphoreType.DMA,
            pltpu.SemaphoreType.DMA,
            pltpu.SemaphoreType.DMA,
        ],
    )
    sc_out = kfn(input_embeddings, emb_table)

    tc_out = pl.pallas_call(
        _tc_body,
        grid=(S // TC_CS, B_TC),
        in_specs=[
            pl.BlockSpec((TC_CS, D), lambda s, b: (s, 0)),
            pl.BlockSpec((1, TC_CS, D), lambda s, b: (b, s, 0)),
        ],
        out_specs=pl.BlockSpec((1, TC_CS, D), lambda s, b: (b, s, 0)),
        out_shape=jax.ShapeDtypeStruct((B_TC, S, D), jnp.float32),
    )(emb_table, input_embeddings)

    return jnp.concatenate([tc_out, sc_out], axis=0)


# speedup vs baseline: 1.4068x; 1.4068x over previous
import functools

import jax
import jax.numpy as jnp
from jax import lax
from jax.experimental import pallas as pl
from jax.experimental.pallas import tpu as pltpu
from jax.experimental.pallas import tpu_sc as plsc

B = 4
S = 8192
D = 1024

_INFO = plsc.get_sparse_core_info()
NC = _INFO.num_cores
NS = _INFO.num_subcores
NW = NC * NS
LANES = 16

ROWS_PER_W = S // NW
C = 32
CL = C * D
N_CHUNKS = ROWS_PER_W // C
BATCH_STRIDE = S * D

B_SC = 1
B_TC = B - B_SC
TC_CS = 512


def _add_chunk(iobuf, tbuf):
    @plsc.parallel_loop(0, C, step=1)
    def _add_row(r):
        @plsc.parallel_loop(0, D, step=LANES, unroll=8)
        def _add(j):
            plsc.addupdate(iobuf.at[r, pl.ds(j, LANES)], tbuf[r, pl.ds(j, LANES)])


def _body(in_hbm, tab_hbm, out_hbm, tbuf, io0, io1, sem_in, sem_out, sem_tab):
    cid = lax.axis_index("c")
    sid = lax.axis_index("s")
    wid = sid * NC + cid
    w_row0 = wid * ROWS_PER_W

    ios = (io0, io1)
    steps = [(c, b) for c in range(N_CHUNKS) for b in range(B_SC)]
    n = len(steps)

    def in_copy(s, buf):
        c, b = steps[s]
        r0 = w_row0 + c * C
        return pltpu.async_copy(in_hbm.at[B_TC + b, pl.ds(r0, C)], buf, sem_in)

    tab_dma = pltpu.async_copy(tab_hbm.at[pl.ds(w_row0, C)], tbuf, sem_tab)
    in_dma = in_copy(0, ios[0])
    out_pending = [None, None]

    for s, (c, b) in enumerate(steps):
        p = s % 2
        if b == 0:
            tab_dma.wait()
        if s + 1 < n:
            if out_pending[1 - p] is not None:
                out_pending[1 - p].wait()
            nxt = in_copy(s + 1, ios[1 - p])
        in_dma.wait()
        in_dma = nxt if s + 1 < n else None
        _add_chunk(ios[p], tbuf)
        if b == B_SC - 1 and c + 1 < N_CHUNKS:
            tab_dma = pltpu.async_copy(
                tab_hbm.at[pl.ds(w_row0 + (c + 1) * C, C)], tbuf, sem_tab
            )
        r0 = w_row0 + c * C
        out_pending[p] = pltpu.async_copy(ios[p], out_hbm.at[b, pl.ds(r0, C)], sem_out)

    for d in out_pending:
        if d is not None:
            d.wait()


def _tc_body(tab_ref, in_ref, out_ref):
    out_ref[...] = in_ref[...] + tab_ref[...][None]


@jax.jit
def kernel(input_embeddings, emb_table):
    kfn = pl.kernel(
        _body,
        out_type=jax.ShapeDtypeStruct((B_SC, S, D), jnp.float32),
        mesh=plsc.VectorSubcoreMesh(core_axis_name="c", subcore_axis_name="s"),
        scratch_types=[
            pltpu.VMEM((C, D), jnp.float32),
            pltpu.VMEM((C, D), jnp.float32),
            pltpu.VMEM((C, D), jnp.float32),
            pltpu.SemaphoreType.DMA,
            pltpu.SemaphoreType.DMA,
            pltpu.SemaphoreType.DMA,
        ],
    )
    sc_out = kfn(input_embeddings, emb_table)

    tc_full = pl.pallas_call(
        _tc_body,
        grid=(S // TC_CS, B_TC),
        in_specs=[
            pl.BlockSpec((TC_CS, D), lambda s, b: (s, 0)),
            pl.BlockSpec((1, TC_CS, D), lambda s, b: (b, s, 0)),
        ],
        out_specs=pl.BlockSpec((1, TC_CS, D), lambda s, b: (b, s, 0)),
        out_shape=jax.ShapeDtypeStruct((B, S, D), jnp.float32),
    )(emb_table, input_embeddings)

    sc_out, tc_full = jax.lax.optimization_barrier((sc_out, tc_full))
    return jax.lax.dynamic_update_slice(tc_full, sc_out, (B_TC, 0, 0))
